# baseline (speedup 1.0000x reference)
import functools

import jax
import jax.numpy as jnp
from jax import lax
from jax.experimental import pallas as pl
from jax.experimental.pallas import tpu as pltpu
from jax.experimental.pallas import tpu_sc as plsc

N = 10000
E = 320000
H = 128
G = 64
C = 10

NC = 2
NS = 16
NW = NC * NS

EW = E // NW
CH = 128
NFULL = EW // CH
TAIL = EW - NFULL * CH
ACH = 96
ANB = 4
ANF = 104
ATAIL = EW - ANF * ACH
ZR = 624
ZREM = N - NS * ZR

_MESH = plsc.VectorSubcoreMesh(core_axis_name="c", subcore_axis_name="s",
                               num_cores=NC, num_subcores=NS)


def _zero_vmem_rows(ref, nrows, width):
    nch = width // 16

    def row(i, carry):
        for j in range(nch):
            ref[i, pl.ds(j * 16, 16)] = jnp.zeros((16,), jnp.float32)
        return carry

    lax.fori_loop(0, nrows, row, 0)


def _deg_body(d_hbm, out_hbm, deg_sh, ones_v, dbuf, dbuf_t, zb, isem, ssem):
    c = lax.axis_index("c")
    s = lax.axis_index("s")
    wid = c * NS + s

    def setrow(i, carry):
        ones_v[i, pl.ds(0, 16)] = jnp.ones((16,), jnp.float32)
        zb[i, pl.ds(0, 16)] = jnp.zeros((16,), jnp.float32)
        return carry

    lax.fori_loop(0, CH, setrow, 0)
    for k, sz in ((0, 128), (128, 128), (256, 128), (384, 128), (512, 112)):
        pltpu.sync_copy(zb.at[pl.ds(0, sz)],
                        deg_sh.at[pl.ds(s * ZR + k, sz)])

    @pl.when(s == NS - 1)
    def _():
        pltpu.sync_copy(zb.at[pl.ds(0, ZREM)],
                        deg_sh.at[pl.ds(NS * ZR, ZREM)])

    plsc.subcore_barrier()

    base = wid * EW

    def issue_idx(i, b):
        pltpu.async_copy(d_hbm.at[pl.ds(base + i * CH, CH)], dbuf.at[b],
                         isem.at[b])

    def wait_idx(b):
        pltpu.make_async_copy(d_hbm.at[pl.ds(base, CH)], dbuf.at[b],
                              isem.at[b]).wait()

    def issue_scatter(b):
        pltpu.async_copy(ones_v, deg_sh.at[dbuf.at[b]], ssem.at[b], add=True)

    def wait_scatter(b):
        pltpu.make_async_copy(ones_v, deg_sh.at[dbuf.at[b]], ssem.at[b]).wait()

    issue_idx(0, 0)
    issue_idx(1, 1)

    def group(g, carry):
        for b in range(3):
            i = g * 3 + b
            wait_idx(b)
            issue_scatter(b)
            nb2 = (b + 2) % 3

            @pl.when(i >= 1)
            def _():
                wait_scatter(nb2)

            @pl.when(i + 2 < NFULL)
            def _():
                issue_idx(i + 2, nb2)

        return carry

    lax.fori_loop(0, NFULL // 3, group, 0)
    wait_scatter((NFULL - 1) % 3)
    pltpu.sync_copy(d_hbm.at[pl.ds(base + NFULL * CH, TAIL)], dbuf_t)
    pltpu.sync_copy(ones_v.at[pl.ds(0, TAIL)], deg_sh.at[dbuf_t], add=True)
    plsc.subcore_barrier()
    pltpu.sync_copy(deg_sh.at[pl.ds(s * ZR, ZR)],
                    out_hbm.at[c, pl.ds(s * ZR, ZR)])

    @pl.when(s == NS - 1)
    def _():
        pltpu.sync_copy(deg_sh.at[pl.ds(NS * ZR, ZREM)],
                        out_hbm.at[c, pl.ds(NS * ZR, ZREM)])


def _deg_partials(d):
    f = pl.kernel(
        _deg_body,
        out_type=jax.ShapeDtypeStruct((NC, N, 16), jnp.float32),
        mesh=_MESH,
        scratch_types=[
            pltpu.VMEM_SHARED((N, 16), jnp.float32),
            pltpu.VMEM((CH, 16), jnp.float32),
            pltpu.VMEM((3, CH), jnp.int32),
            pltpu.VMEM((TAIL,), jnp.int32),
            pltpu.VMEM((CH, 16), jnp.float32),
            pltpu.SemaphoreType.DMA((3,)),
            pltpu.SemaphoreType.DMA((3,)),
        ],
    )
    return f(d)


def _agg_body(y_hbm, s_hbm, d_hbm, out_hbm, acc_sh, sbuf, dbuf, rows,
              sbuf_t, dbuf_t, isem, gsem, ssem, tsem):
    c = lax.axis_index("c")
    s = lax.axis_index("s")
    wid = c * NS + s

    def zrow(i, carry):
        for j in range(H // 16):
            rows[0, i, pl.ds(j * 16, 16)] = jnp.zeros((16,), jnp.float32)
        return carry

    lax.fori_loop(0, 128, zrow, 0)
    for k, sz in ((0, 128), (128, 128), (256, 128), (384, 128), (512, 112)):
        pltpu.sync_copy(rows.at[0, pl.ds(0, sz)],
                        acc_sh.at[pl.ds(s * ZR + k, sz)])

    @pl.when(s == NS - 1)
    def _():
        pltpu.sync_copy(rows.at[0, pl.ds(0, ZREM)],
                        acc_sh.at[pl.ds(NS * ZR, ZREM)])

    plsc.subcore_barrier()

    base = wid * EW

    def issue_idx(i, b):
        off = base + i * ACH
        pltpu.async_copy(s_hbm.at[pl.ds(off, ACH)], sbuf.at[b], isem.at[b])
        pltpu.async_copy(d_hbm.at[pl.ds(off, ACH)], dbuf.at[b], isem.at[b])

    def wait_idx(b):
        pltpu.make_async_copy(s_hbm.at[pl.ds(base, ACH)], sbuf.at[b],
                              isem.at[b]).wait()
        pltpu.make_async_copy(d_hbm.at[pl.ds(base, ACH)], dbuf.at[b],
                              isem.at[b]).wait()

    def issue_gather(b):
        pltpu.async_copy(y_hbm.at[sbuf.at[b]], rows.at[b], gsem.at[b])

    def wait_gather(b):
        pltpu.make_async_copy(y_hbm.at[sbuf.at[b]], rows.at[b],
                              gsem.at[b]).wait()

    def issue_scatter(b):
        pltpu.async_copy(rows.at[b], acc_sh.at[dbuf.at[b]], ssem.at[b],
                         add=True)

    def wait_scatter(b):
        pltpu.make_async_copy(rows.at[b], acc_sh.at[dbuf.at[b]],
                              ssem.at[b]).wait()

    issue_idx(0, 0)
    issue_idx(1, 1)
    wait_idx(0)
    issue_gather(0)

    def group(g, carry):
        for b in range(ANB):
            i = g * ANB + b
            nb1 = (b + 1) % ANB
            nb2 = (b + 2) % ANB
            wait_gather(b)
            issue_scatter(b)

            @pl.when(i >= 2)
            def _():
                wait_scatter(nb2)

            @pl.when(i + 2 < ANF)
            def _():
                issue_idx(i + 2, nb2)

            @pl.when(i + 1 < ANF)
            def _():
                wait_idx(nb1)
                issue_gather(nb1)

        return carry

    lax.fori_loop(0, ANF // ANB, group, 0)
    wait_scatter((ANF - 2) % ANB)
    wait_scatter((ANF - 1) % ANB)

    off = base + ANF * ACH
    pltpu.sync_copy(s_hbm.at[pl.ds(off, ATAIL)], sbuf_t)
    pltpu.sync_copy(d_hbm.at[pl.ds(off, ATAIL)], dbuf_t)
    pltpu.async_copy(y_hbm.at[sbuf_t], rows.at[0, pl.ds(0, ATAIL)], tsem).wait()
    pltpu.sync_copy(rows.at[0, pl.ds(0, ATAIL)], acc_sh.at[dbuf_t], add=True)
    plsc.subcore_barrier()
    pltpu.sync_copy(acc_sh.at[pl.ds(s * ZR, ZR)],
                    out_hbm.at[c, pl.ds(s * ZR, ZR)])

    @pl.when(s == NS - 1)
    def _():
        pltpu.sync_copy(acc_sh.at[pl.ds(NS * ZR, ZREM)],
                        out_hbm.at[c, pl.ds(NS * ZR, ZREM)])


def _agg_partials(y, s, d):
    f = pl.kernel(
        _agg_body,
        out_type=jax.ShapeDtypeStruct((NC, N, H), jnp.float32),
        mesh=_MESH,
        scratch_types=[
            pltpu.VMEM_SHARED((N, H), jnp.float32),
            pltpu.VMEM((ANB, ACH), jnp.int32),
            pltpu.VMEM((ANB, ACH), jnp.int32),
            pltpu.VMEM((ANB, ACH, H), jnp.float32),
            pltpu.VMEM((ATAIL,), jnp.int32),
            pltpu.VMEM((ATAIL,), jnp.int32),
            pltpu.SemaphoreType.DMA((ANB,)),
            pltpu.SemaphoreType.DMA((ANB,)),
            pltpu.SemaphoreType.DMA((ANB,)),
            pltpu.SemaphoreType.DMA,
        ],
    )
    return f(y, s, d)


PCH = 64
NPCH = (N + PCH - 1) // PCH
PTAIL = N - (NPCH - 1) * PCH


def _pool_body(h_hbm, b_hbm, mx_hbm, sm_hbm, cn_hbm,
               rowbuf, bbuf, rowbuf_t, bbuf_t, mx, sm, cn):
    c = lax.axis_index("c")
    s = lax.axis_index("s")
    wid = c * NS + s

    neg_inf = jnp.full((16,), -jnp.inf, dtype=jnp.float32)

    def initrow(i, carry):
        for j in range(H // 16):
            mx[i, pl.ds(j * 16, 16)] = neg_inf
            sm[i, pl.ds(j * 16, 16)] = jnp.zeros((16,), jnp.float32)
        cn[i, pl.ds(0, 16)] = jnp.zeros((16,), jnp.float32)
        return carry

    lax.fori_loop(0, G, initrow, 0)

    ones16 = jnp.ones((16,), jnp.float32)

    def accum_row(rb, bb, i):
        g = bb[pl.ds(i, 16)][0]
        plsc.addupdate(cn.at[g], ones16)
        for j in range(H // 16):
            r = rb[i, pl.ds(j * 16, 16)]
            plsc.addupdate(sm.at[g, pl.ds(j * 16, 16)], r)
            m = mx[g, pl.ds(j * 16, 16)]
            mx[g, pl.ds(j * 16, 16)] = jnp.maximum(m, r)

    def do_chunk(jj, carry):
        k = wid + jj * NW

        @pl.when(k < NPCH - 1)
        def _():
            pltpu.sync_copy(h_hbm.at[pl.ds(k * PCH, PCH)], rowbuf)
            pltpu.sync_copy(b_hbm.at[pl.ds(k * PCH, PCH)], bbuf.at[pl.ds(0, PCH)])

            def row(i, cc):
                accum_row(rowbuf, bbuf, i)
                return cc

            lax.fori_loop(0, PCH, row, 0)

        @pl.when(k == NPCH - 1)
        def _():
            pltpu.sync_copy(h_hbm.at[pl.ds((NPCH - 1) * PCH, PTAIL)], rowbuf_t)
            pltpu.sync_copy(b_hbm.at[pl.ds((NPCH - 1) * PCH, PTAIL)],
                            bbuf_t.at[pl.ds(0, PTAIL)])

            def row(i, cc):
                accum_row(rowbuf_t, bbuf_t, i)
                return cc

            lax.fori_loop(0, PTAIL, row, 0)

        return carry

    lax.fori_loop(0, (NPCH + NW - 1) // NW, do_chunk, 0)

    pltpu.sync_copy(mx, mx_hbm.at[wid])
    pltpu.sync_copy(sm, sm_hbm.at[wid])
    pltpu.sync_copy(cn, cn_hbm.at[wid])


def _pool_partials(h, batch):
    f = pl.kernel(
        _pool_body,
        out_type=(
            jax.ShapeDtypeStruct((NW, G, H), jnp.float32),
            jax.ShapeDtypeStruct((NW, G, H), jnp.float32),
            jax.ShapeDtypeStruct((NW, G, 16), jnp.float32),
        ),
        mesh=_MESH,
        scratch_types=[
            pltpu.VMEM((PCH, H), jnp.float32),
            pltpu.VMEM((PCH + 16,), jnp.int32),
            pltpu.VMEM((PTAIL, H), jnp.float32),
            pltpu.VMEM((PTAIL + 16,), jnp.int32),
            pltpu.VMEM((G, H), jnp.float32),
            pltpu.VMEM((G, H), jnp.float32),
            pltpu.VMEM((G, 16), jnp.float32),
        ],
    )
    return f(h, batch)


RB = 1000


def _dinv_from_parts(deg_parts):
    deg = 1.0 + deg_parts[0, :, 0] + deg_parts[1, :, 0]
    return 1.0 / jnp.sqrt(deg)


def _xw_body(x_ref, w_ref, degp_ref, y_ref):
    dinv = _dinv_from_parts(degp_ref[...])
    xw = jnp.dot(x_ref[...], w_ref[...], preferred_element_type=jnp.float32)
    y_ref[...] = dinv[:, None] * xw


def _xw_scaled(x, w, deg_parts):
    fin = x.shape[1]
    return pl.pallas_call(
        _xw_body,
        grid=(N // RB,),
        in_specs=[
            pl.BlockSpec((RB, fin), lambda i: (i, 0)),
            pl.BlockSpec((fin, H), lambda i: (0, 0)),
            pl.BlockSpec((NC, RB, 16), lambda i: (0, i, 0)),
        ],
        out_specs=pl.BlockSpec((RB, H), lambda i: (i, 0)),
        out_shape=jax.ShapeDtypeStruct((N, H), jnp.float32),
    )(x, w, deg_parts)


def _combine_mm_body(aggp_ref, y_ref, degp_ref, b_ref, w_ref, out_ref):
    dinv = _dinv_from_parts(degp_ref[...])
    h = aggp_ref[0] + aggp_ref[1] + y_ref[...]
    h = jax.nn.relu(dinv[:, None] * h + b_ref[...])
    hw = jnp.dot(h, w_ref[...], preferred_element_type=jnp.float32)
    out_ref[...] = dinv[:, None] * hw


def _combine_matmul(aggp, y, deg_parts, b, w):
    return pl.pallas_call(
        _combine_mm_body,
        grid=(N // RB,),
        in_specs=[
            pl.BlockSpec((NC, RB, H), lambda i: (0, i, 0)),
            pl.BlockSpec((RB, H), lambda i: (i, 0)),
            pl.BlockSpec((NC, RB, 16), lambda i: (0, i, 0)),
            pl.BlockSpec((1, H), lambda i: (0, 0)),
            pl.BlockSpec((H, H), lambda i: (0, 0)),
        ],
        out_specs=pl.BlockSpec((RB, H), lambda i: (i, 0)),
        out_shape=jax.ShapeDtypeStruct((N, H), jnp.float32),
    )(aggp, y, deg_parts, b, w)


def _combine_body(aggp_ref, y_ref, degp_ref, b_ref, out_ref):
    dinv = _dinv_from_parts(degp_ref[...])
    h = aggp_ref[0] + aggp_ref[1] + y_ref[...]
    out_ref[...] = jax.nn.relu(dinv[:, None] * h + b_ref[...])


def _combine(aggp, y, deg_parts, b):
    return pl.pallas_call(
        _combine_body,
        grid=(N // RB,),
        in_specs=[
            pl.BlockSpec((NC, RB, H), lambda i: (0, i, 0)),
            pl.BlockSpec((RB, H), lambda i: (i, 0)),
            pl.BlockSpec((NC, RB, 16), lambda i: (0, i, 0)),
            pl.BlockSpec((1, H), lambda i: (0, 0)),
        ],
        out_specs=pl.BlockSpec((RB, H), lambda i: (i, 0)),
        out_shape=jax.ShapeDtypeStruct((N, H), jnp.float32),
    )(aggp, y, deg_parts, b)


def _final_body(mxp_ref, smp_ref, cnp_ref, w_ref, b_ref, out_ref):
    mx = jnp.max(mxp_ref[...], axis=0)
    sm = jnp.sum(smp_ref[...], axis=0)
    cnt = jnp.sum(cnp_ref[..., 0], axis=0)
    mean = sm / jnp.maximum(cnt, 1.0)[:, None]
    z = jnp.concatenate([mx, mean], axis=1)
    z = jnp.dot(z, w_ref[...], preferred_element_type=jnp.float32) + b_ref[...]
    m = jnp.max(z, axis=1, keepdims=True)
    lse = jnp.log(jnp.sum(jnp.exp(z - m), axis=1, keepdims=True)) + m
    out_ref[...] = z - lse


def _final(mxp, smp, cnp, lin_W, lin_b):
    return pl.pallas_call(
        _final_body,
        out_shape=jax.ShapeDtypeStruct((G, C), jnp.float32),
    )(mxp, smp, cnp, lin_W, lin_b.reshape(1, C))


def kernel(x, edge_index, batch, W1, b1, W2, b2, lin_W, lin_b):
    s = edge_index[0]
    d = edge_index[1]
    deg_parts = _deg_partials(d)
    y1 = _xw_scaled(x, W1, deg_parts)
    agg1 = _agg_partials(y1, s, d)
    y2 = _combine_matmul(agg1, y1, deg_parts, b1.reshape(1, H), W2)
    agg2 = _agg_partials(y2, s, d)
    h2 = _combine(agg2, y2, deg_parts, b2.reshape(1, H))
    mxp, smp, cnp = _pool_partials(h2, batch)
    return _final(mxp, smp, cnp, lin_W, lin_b)


# speedup vs baseline: 1.0562x; 1.0562x over previous
import functools

import jax
import jax.numpy as jnp
from jax import lax
from jax.experimental import pallas as pl
from jax.experimental.pallas import tpu as pltpu
from jax.experimental.pallas import tpu_sc as plsc

N = 10000
E = 320000
H = 128
G = 64
C = 10

NC = 2
NS = 16
NW = NC * NS

EW = E // NW
CH = 128
NFULL = EW // CH
TAIL = EW - NFULL * CH
ZR = 624
ZREM = N - NS * ZR

_MESH = plsc.VectorSubcoreMesh(core_axis_name="c", subcore_axis_name="s",
                               num_cores=NC, num_subcores=NS)


def _zero_vmem_rows(ref, nrows, width):
    nch = width // 16

    def row(i, carry):
        for j in range(nch):
            ref[i, pl.ds(j * 16, 16)] = jnp.zeros((16,), jnp.float32)
        return carry

    lax.fori_loop(0, nrows, row, 0)


def _deg_body(d_hbm, out_hbm, deg_sh, ones_v, dbuf, dbuf_t, zb, isem, ssem):
    c = lax.axis_index("c")
    s = lax.axis_index("s")
    wid = c * NS + s

    def setrow(i, carry):
        ones_v[i, pl.ds(0, 16)] = jnp.ones((16,), jnp.float32)
        zb[i, pl.ds(0, 16)] = jnp.zeros((16,), jnp.float32)
        return carry

    lax.fori_loop(0, CH, setrow, 0)
    for k, sz in ((0, 128), (128, 128), (256, 128), (384, 128), (512, 112)):
        pltpu.sync_copy(zb.at[pl.ds(0, sz)],
                        deg_sh.at[pl.ds(s * ZR + k, sz)])

    @pl.when(s == NS - 1)
    def _():
        pltpu.sync_copy(zb.at[pl.ds(0, ZREM)],
                        deg_sh.at[pl.ds(NS * ZR, ZREM)])

    plsc.subcore_barrier()

    base = wid * EW

    def issue_idx(i, b):
        pltpu.async_copy(d_hbm.at[pl.ds(base + i * CH, CH)], dbuf.at[b],
                         isem.at[b])

    def wait_idx(b):
        pltpu.make_async_copy(d_hbm.at[pl.ds(base, CH)], dbuf.at[b],
                              isem.at[b]).wait()

    def issue_scatter(b):
        pltpu.async_copy(ones_v, deg_sh.at[dbuf.at[b]], ssem.at[b], add=True)

    def wait_scatter(b):
        pltpu.make_async_copy(ones_v, deg_sh.at[dbuf.at[b]], ssem.at[b]).wait()

    issue_idx(0, 0)
    issue_idx(1, 1)

    def group(g, carry):
        for b in range(3):
            i = g * 3 + b
            wait_idx(b)
            issue_scatter(b)
            nb2 = (b + 2) % 3

            @pl.when(i >= 1)
            def _():
                wait_scatter(nb2)

            @pl.when(i + 2 < NFULL)
            def _():
                issue_idx(i + 2, nb2)

        return carry

    lax.fori_loop(0, NFULL // 3, group, 0)
    wait_scatter((NFULL - 1) % 3)
    pltpu.sync_copy(d_hbm.at[pl.ds(base + NFULL * CH, TAIL)], dbuf_t)
    pltpu.sync_copy(ones_v.at[pl.ds(0, TAIL)], deg_sh.at[dbuf_t], add=True)
    plsc.subcore_barrier()
    pltpu.sync_copy(deg_sh.at[pl.ds(s * ZR, ZR)],
                    out_hbm.at[c, pl.ds(s * ZR, ZR)])

    @pl.when(s == NS - 1)
    def _():
        pltpu.sync_copy(deg_sh.at[pl.ds(NS * ZR, ZREM)],
                        out_hbm.at[c, pl.ds(NS * ZR, ZREM)])


def _deg_partials(d):
    f = pl.kernel(
        _deg_body,
        out_type=jax.ShapeDtypeStruct((NC, N, 16), jnp.float32),
        mesh=_MESH,
        scratch_types=[
            pltpu.VMEM_SHARED((N, 16), jnp.float32),
            pltpu.VMEM((CH, 16), jnp.float32),
            pltpu.VMEM((3, CH), jnp.int32),
            pltpu.VMEM((TAIL,), jnp.int32),
            pltpu.VMEM((CH, 16), jnp.float32),
            pltpu.SemaphoreType.DMA((3,)),
            pltpu.SemaphoreType.DMA((3,)),
        ],
    )
    return f(d)


def _agg_body(y_hbm, s_hbm, d_hbm, out_hbm, acc_sh, sbuf, dbuf, rows,
              sbuf_t, dbuf_t, isem, gsem, ssem, tsem):
    c = lax.axis_index("c")
    s = lax.axis_index("s")
    wid = c * NS + s

    def zrow(i, carry):
        for j in range(H // 16):
            rows[0, i, pl.ds(j * 16, 16)] = jnp.zeros((16,), jnp.float32)
        return carry

    lax.fori_loop(0, 128, zrow, 0)
    for k, sz in ((0, 128), (128, 128), (256, 128), (384, 128), (512, 112)):
        pltpu.sync_copy(rows.at[0, pl.ds(0, sz)],
                        acc_sh.at[pl.ds(s * ZR + k, sz)])

    @pl.when(s == NS - 1)
    def _():
        pltpu.sync_copy(rows.at[0, pl.ds(0, ZREM)],
                        acc_sh.at[pl.ds(NS * ZR, ZREM)])

    plsc.subcore_barrier()

    base = wid * EW

    def issue_idx(i, b):
        off = base + i * CH
        pltpu.async_copy(s_hbm.at[pl.ds(off, CH)], sbuf.at[b], isem.at[b])
        pltpu.async_copy(d_hbm.at[pl.ds(off, CH)], dbuf.at[b], isem.at[b])

    def wait_idx(b):
        pltpu.make_async_copy(s_hbm.at[pl.ds(base, CH)], sbuf.at[b],
                              isem.at[b]).wait()
        pltpu.make_async_copy(d_hbm.at[pl.ds(base, CH)], dbuf.at[b],
                              isem.at[b]).wait()

    def issue_gather(b):
        pltpu.async_copy(y_hbm.at[sbuf.at[b]], rows.at[b], gsem.at[b])

    def wait_gather(b):
        pltpu.make_async_copy(y_hbm.at[sbuf.at[b]], rows.at[b],
                              gsem.at[b]).wait()

    def issue_scatter(b):
        pltpu.async_copy(rows.at[b], acc_sh.at[dbuf.at[b]], ssem.at[b],
                         add=True)

    def wait_scatter(b):
        pltpu.make_async_copy(rows.at[b], acc_sh.at[dbuf.at[b]],
                              ssem.at[b]).wait()

    issue_idx(0, 0)
    issue_idx(1, 1)
    wait_idx(0)
    issue_gather(0)

    def group(g, carry):
        for b in range(3):
            i = g * 3 + b
            nb1 = (b + 1) % 3
            nb2 = (b + 2) % 3
            wait_gather(b)
            issue_scatter(b)

            @pl.when(i >= 1)
            def _():
                wait_scatter(nb2)

            @pl.when(i + 2 < NFULL)
            def _():
                issue_idx(i + 2, nb2)

            @pl.when(i + 1 < NFULL)
            def _():
                wait_idx(nb1)
                issue_gather(nb1)

        return carry

    lax.fori_loop(0, NFULL // 3, group, 0)
    wait_scatter((NFULL - 1) % 3)

    off = base + NFULL * CH
    pltpu.sync_copy(s_hbm.at[pl.ds(off, TAIL)], sbuf_t)
    pltpu.sync_copy(d_hbm.at[pl.ds(off, TAIL)], dbuf_t)
    pltpu.async_copy(y_hbm.at[sbuf_t], rows.at[0, pl.ds(0, TAIL)], tsem).wait()
    pltpu.sync_copy(rows.at[0, pl.ds(0, TAIL)], acc_sh.at[dbuf_t], add=True)
    plsc.subcore_barrier()
    pltpu.sync_copy(acc_sh.at[pl.ds(s * ZR, ZR)],
                    out_hbm.at[c, pl.ds(s * ZR, ZR)])

    @pl.when(s == NS - 1)
    def _():
        pltpu.sync_copy(acc_sh.at[pl.ds(NS * ZR, ZREM)],
                        out_hbm.at[c, pl.ds(NS * ZR, ZREM)])


def _agg_partials(y, s, d):
    f = pl.kernel(
        _agg_body,
        out_type=jax.ShapeDtypeStruct((NC, N, H), jnp.float32),
        mesh=_MESH,
        scratch_types=[
            pltpu.VMEM_SHARED((N, H), jnp.float32),
            pltpu.VMEM((3, CH), jnp.int32),
            pltpu.VMEM((3, CH), jnp.int32),
            pltpu.VMEM((3, CH, H), jnp.float32),
            pltpu.VMEM((TAIL,), jnp.int32),
            pltpu.VMEM((TAIL,), jnp.int32),
            pltpu.SemaphoreType.DMA((3,)),
            pltpu.SemaphoreType.DMA((3,)),
            pltpu.SemaphoreType.DMA((3,)),
            pltpu.SemaphoreType.DMA,
        ],
    )
    return f(y, s, d)


PCH = 64
NPCH = (N + PCH - 1) // PCH
PTAIL = N - (NPCH - 1) * PCH


def _pool_body(aggp_hbm, y_hbm, dv_hbm, b2_hbm, b_hbm, mx_hbm, sm_hbm,
               cn_hbm, a0, a1, yb, dv, bbuf, a0t, a1t, ybt, dvt,
               bbuf_t, b2v, mx, sm, cn, sem):
    c = lax.axis_index("c")
    s = lax.axis_index("s")
    wid = c * NS + s

    pltpu.sync_copy(b2_hbm, b2v)

    neg_inf = jnp.full((16,), -jnp.inf, dtype=jnp.float32)

    def initrow(i, carry):
        for j in range(H // 16):
            mx[i, pl.ds(j * 16, 16)] = neg_inf
            sm[i, pl.ds(j * 16, 16)] = jnp.zeros((16,), jnp.float32)
        cn[i, pl.ds(0, 16)] = jnp.zeros((16,), jnp.float32)
        return carry

    lax.fori_loop(0, G, initrow, 0)

    ones16 = jnp.ones((16,), jnp.float32)

    def fetch(k, n, a0b, a1b, ybb, dvb, bb):
        r0 = k * PCH
        cps = [
            (aggp_hbm.at[0, pl.ds(r0, n)], a0b),
            (aggp_hbm.at[1, pl.ds(r0, n)], a1b),
            (y_hbm.at[pl.ds(r0, n)], ybb),
            (dv_hbm.at[pl.ds(r0, n)], dvb),
            (b_hbm.at[pl.ds(r0, n)], bb.at[pl.ds(0, n)]),
        ]
        for sr, dst in cps:
            pltpu.async_copy(sr, dst, sem)
        for sr, dst in cps:
            pltpu.make_async_copy(sr, dst, sem).wait()

    def accum_row(a0b, a1b, ybb, dvb, bb, i):
        g = bb[pl.ds(i, 16)][0]
        di = dvb[i, pl.ds(0, 16)]
        plsc.addupdate(cn.at[g], ones16)
        for j in range(H // 16):
            jds = pl.ds(j * 16, 16)
            h = di * (a0b[i, jds] + a1b[i, jds] + ybb[i, jds]) + b2v[jds]
            h = jnp.maximum(h, 0.0)
            plsc.addupdate(sm.at[g, jds], h)
            m = mx[g, jds]
            mx[g, jds] = jnp.maximum(m, h)

    def do_chunk(jj, carry):
        k = wid + jj * NW

        @pl.when(k < NPCH - 1)
        def _():
            fetch(k, PCH, a0, a1, yb, dv, bbuf)

            def row(i, cc):
                accum_row(a0, a1, yb, dv, bbuf, i)
                return cc

            lax.fori_loop(0, PCH, row, 0)

        @pl.when(k == NPCH - 1)
        def _():
            fetch(k, PTAIL, a0t, a1t, ybt, dvt, bbuf_t)

            def row(i, cc):
                accum_row(a0t, a1t, ybt, dvt, bbuf_t, i)
                return cc

            lax.fori_loop(0, PTAIL, row, 0)

        return carry

    lax.fori_loop(0, (NPCH + NW - 1) // NW, do_chunk, 0)

    pltpu.sync_copy(mx, mx_hbm.at[wid])
    pltpu.sync_copy(sm, sm_hbm.at[wid])
    pltpu.sync_copy(cn, cn_hbm.at[wid])


def _pool_partials(aggp, y, dinv16, b2, batch):
    f = pl.kernel(
        _pool_body,
        out_type=(
            jax.ShapeDtypeStruct((NW, G, H), jnp.float32),
            jax.ShapeDtypeStruct((NW, G, H), jnp.float32),
            jax.ShapeDtypeStruct((NW, G, 16), jnp.float32),
        ),
        mesh=_MESH,
        scratch_types=[
            pltpu.VMEM((PCH, H), jnp.float32),
            pltpu.VMEM((PCH, H), jnp.float32),
            pltpu.VMEM((PCH, H), jnp.float32),
            pltpu.VMEM((PCH, 16), jnp.float32),
            pltpu.VMEM((PCH + 16,), jnp.int32),
            pltpu.VMEM((PTAIL, H), jnp.float32),
            pltpu.VMEM((PTAIL, H), jnp.float32),
            pltpu.VMEM((PTAIL, H), jnp.float32),
            pltpu.VMEM((PTAIL, 16), jnp.float32),
            pltpu.VMEM((PTAIL + 16,), jnp.int32),
            pltpu.VMEM((H,), jnp.float32),
            pltpu.VMEM((G, H), jnp.float32),
            pltpu.VMEM((G, H), jnp.float32),
            pltpu.VMEM((G, 16), jnp.float32),
            pltpu.SemaphoreType.DMA,
        ],
    )
    return f(aggp, y, dinv16, b2, batch)


RB = 1000


def _dinv_from_parts(deg_parts):
    deg = 1.0 + deg_parts[0, :, 0] + deg_parts[1, :, 0]
    return 1.0 / jnp.sqrt(deg)


def _xw_body(x_ref, w_ref, degp_ref, y_ref, dv_ref):
    dinv = _dinv_from_parts(degp_ref[...])
    xw = jnp.dot(x_ref[...], w_ref[...], preferred_element_type=jnp.float32)
    y_ref[...] = dinv[:, None] * xw
    dv_ref[...] = jnp.broadcast_to(dinv[:, None], (RB, 16))


def _xw_scaled(x, w, deg_parts):
    fin = x.shape[1]
    return pl.pallas_call(
        _xw_body,
        grid=(N // RB,),
        in_specs=[
            pl.BlockSpec((RB, fin), lambda i: (i, 0)),
            pl.BlockSpec((fin, H), lambda i: (0, 0)),
            pl.BlockSpec((NC, RB, 16), lambda i: (0, i, 0)),
        ],
        out_specs=(pl.BlockSpec((RB, H), lambda i: (i, 0)),
                   pl.BlockSpec((RB, 16), lambda i: (i, 0))),
        out_shape=(jax.ShapeDtypeStruct((N, H), jnp.float32),
                   jax.ShapeDtypeStruct((N, 16), jnp.float32)),
    )(x, w, deg_parts)


def _combine_mm_body(aggp_ref, y_ref, degp_ref, b_ref, w_ref, out_ref):
    dinv = _dinv_from_parts(degp_ref[...])
    h = aggp_ref[0] + aggp_ref[1] + y_ref[...]
    h = jax.nn.relu(dinv[:, None] * h + b_ref[...])
    hw = jnp.dot(h, w_ref[...], preferred_element_type=jnp.float32)
    out_ref[...] = dinv[:, None] * hw


def _combine_matmul(aggp, y, deg_parts, b, w):
    return pl.pallas_call(
        _combine_mm_body,
        grid=(N // RB,),
        in_specs=[
            pl.BlockSpec((NC, RB, H), lambda i: (0, i, 0)),
            pl.BlockSpec((RB, H), lambda i: (i, 0)),
            pl.BlockSpec((NC, RB, 16), lambda i: (0, i, 0)),
            pl.BlockSpec((1, H), lambda i: (0, 0)),
            pl.BlockSpec((H, H), lambda i: (0, 0)),
        ],
        out_specs=pl.BlockSpec((RB, H), lambda i: (i, 0)),
        out_shape=jax.ShapeDtypeStruct((N, H), jnp.float32),
    )(aggp, y, deg_parts, b, w)


def _final_body(mxp_ref, smp_ref, cnp_ref, w_ref, b_ref, out_ref):
    mx = jnp.max(mxp_ref[...], axis=0)
    sm = jnp.sum(smp_ref[...], axis=0)
    cnt = jnp.sum(cnp_ref[..., 0], axis=0)
    mean = sm / jnp.maximum(cnt, 1.0)[:, None]
    z = jnp.concatenate([mx, mean], axis=1)
    z = jnp.dot(z, w_ref[...], preferred_element_type=jnp.float32) + b_ref[...]
    m = jnp.max(z, axis=1, keepdims=True)
    lse = jnp.log(jnp.sum(jnp.exp(z - m), axis=1, keepdims=True)) + m
    out_ref[...] = z - lse


def _final(mxp, smp, cnp, lin_W, lin_b):
    return pl.pallas_call(
        _final_body,
        out_shape=jax.ShapeDtypeStruct((G, C), jnp.float32),
    )(mxp, smp, cnp, lin_W, lin_b.reshape(1, C))


def kernel(x, edge_index, batch, W1, b1, W2, b2, lin_W, lin_b):
    s = edge_index[0]
    d = edge_index[1]
    deg_parts = _deg_partials(d)
    y1, dinv16 = _xw_scaled(x, W1, deg_parts)
    agg1 = _agg_partials(y1, s, d)
    y2 = _combine_matmul(agg1, y1, deg_parts, b1.reshape(1, H), W2)
    agg2 = _agg_partials(y2, s, d)
    mxp, smp, cnp = _pool_partials(agg2, y2, dinv16, b2, batch)
    return _final(mxp, smp, cnp, lin_W, lin_b)


# speedup vs baseline: 1.0766x; 1.0193x over previous
import functools

import jax
import jax.numpy as jnp
from jax import lax
from jax.experimental import pallas as pl
from jax.experimental.pallas import tpu as pltpu
from jax.experimental.pallas import tpu_sc as plsc

N = 10000
E = 320000
H = 128
G = 64
C = 10

NC = 2
NS = 16
NW = NC * NS

EW = E // NW
CH = 128
NFULL = EW // CH
TAIL = EW - NFULL * CH
ZR = 624
ZREM = N - NS * ZR

_MESH = plsc.VectorSubcoreMesh(core_axis_name="c", subcore_axis_name="s",
                               num_cores=NC, num_subcores=NS)


def _zero_vmem_rows(ref, nrows, width):
    nch = width // 16

    def row(i, carry):
        for j in range(nch):
            ref[i, pl.ds(j * 16, 16)] = jnp.zeros((16,), jnp.float32)
        return carry

    lax.fori_loop(0, nrows, row, 0)


def _deg_body(d_hbm, out_hbm, deg_sh, ones_v, dbuf, dbuf_t, zb, isem, ssem):
    c = lax.axis_index("c")
    s = lax.axis_index("s")
    wid = c * NS + s

    def setrow(i, carry):
        ones_v[i, pl.ds(0, 16)] = jnp.ones((16,), jnp.float32)
        zb[i, pl.ds(0, 16)] = jnp.zeros((16,), jnp.float32)
        return carry

    lax.fori_loop(0, CH, setrow, 0)
    for k, sz in ((0, 128), (128, 128), (256, 128), (384, 128), (512, 112)):
        pltpu.sync_copy(zb.at[pl.ds(0, sz)],
                        deg_sh.at[pl.ds(s * ZR + k, sz)])

    @pl.when(s == NS - 1)
    def _():
        pltpu.sync_copy(zb.at[pl.ds(0, ZREM)],
                        deg_sh.at[pl.ds(NS * ZR, ZREM)])

    plsc.subcore_barrier()

    base = wid * EW

    def issue_idx(i, b):
        pltpu.async_copy(d_hbm.at[pl.ds(base + i * CH, CH)], dbuf.at[b],
                         isem.at[b])

    def wait_idx(b):
        pltpu.make_async_copy(d_hbm.at[pl.ds(base, CH)], dbuf.at[b],
                              isem.at[b]).wait()

    def issue_scatter(b):
        pltpu.async_copy(ones_v, deg_sh.at[dbuf.at[b]], ssem.at[b], add=True)

    def wait_scatter(b):
        pltpu.make_async_copy(ones_v, deg_sh.at[dbuf.at[b]], ssem.at[b]).wait()

    issue_idx(0, 0)
    issue_idx(1, 1)

    def group(g, carry):
        for b in range(3):
            i = g * 3 + b
            wait_idx(b)
            issue_scatter(b)
            nb2 = (b + 2) % 3

            @pl.when(i >= 1)
            def _():
                wait_scatter(nb2)

            @pl.when(i + 2 < NFULL)
            def _():
                issue_idx(i + 2, nb2)

        return carry

    lax.fori_loop(0, NFULL // 3, group, 0)
    wait_scatter((NFULL - 1) % 3)
    pltpu.sync_copy(d_hbm.at[pl.ds(base + NFULL * CH, TAIL)], dbuf_t)
    pltpu.sync_copy(ones_v.at[pl.ds(0, TAIL)], deg_sh.at[dbuf_t], add=True)
    plsc.subcore_barrier()
    pltpu.sync_copy(deg_sh.at[pl.ds(s * ZR, ZR)],
                    out_hbm.at[c, pl.ds(s * ZR, ZR)])

    @pl.when(s == NS - 1)
    def _():
        pltpu.sync_copy(deg_sh.at[pl.ds(NS * ZR, ZREM)],
                        out_hbm.at[c, pl.ds(NS * ZR, ZREM)])


def _deg_partials(d):
    f = pl.kernel(
        _deg_body,
        out_type=jax.ShapeDtypeStruct((NC, N, 16), jnp.float32),
        mesh=_MESH,
        scratch_types=[
            pltpu.VMEM_SHARED((N, 16), jnp.float32),
            pltpu.VMEM((CH, 16), jnp.float32),
            pltpu.VMEM((3, CH), jnp.int32),
            pltpu.VMEM((TAIL,), jnp.int32),
            pltpu.VMEM((CH, 16), jnp.float32),
            pltpu.SemaphoreType.DMA((3,)),
            pltpu.SemaphoreType.DMA((3,)),
        ],
    )
    return f(d)


def _agg_body(y_hbm, s_hbm, d_hbm, out_hbm, acc_sh, sbuf, dbuf, rows,
              sbuf_t, dbuf_t, isem, gsem, ssem, tsem):
    c = lax.axis_index("c")
    s = lax.axis_index("s")
    wid = c * NS + s

    def zrow(i, carry):
        for j in range(H // 16):
            rows[0, i, pl.ds(j * 16, 16)] = jnp.zeros((16,), jnp.float32)
        return carry

    lax.fori_loop(0, 128, zrow, 0)
    for k, sz in ((0, 128), (128, 128), (256, 128), (384, 128), (512, 112)):
        pltpu.sync_copy(rows.at[0, pl.ds(0, sz)],
                        acc_sh.at[pl.ds(s * ZR + k, sz)])

    @pl.when(s == NS - 1)
    def _():
        pltpu.sync_copy(rows.at[0, pl.ds(0, ZREM)],
                        acc_sh.at[pl.ds(NS * ZR, ZREM)])

    plsc.subcore_barrier()

    base = wid * EW

    def issue_idx(i, b):
        off = base + i * CH
        pltpu.async_copy(s_hbm.at[pl.ds(off, CH)], sbuf.at[b], isem.at[b])
        pltpu.async_copy(d_hbm.at[pl.ds(off, CH)], dbuf.at[b], isem.at[b])

    def wait_idx(b):
        pltpu.make_async_copy(s_hbm.at[pl.ds(base, CH)], sbuf.at[b],
                              isem.at[b]).wait()
        pltpu.make_async_copy(d_hbm.at[pl.ds(base, CH)], dbuf.at[b],
                              isem.at[b]).wait()

    def issue_gather(b):
        pltpu.async_copy(y_hbm.at[sbuf.at[b]], rows.at[b], gsem.at[b])

    def wait_gather(b):
        pltpu.make_async_copy(y_hbm.at[sbuf.at[b]], rows.at[b],
                              gsem.at[b]).wait()

    def issue_scatter(b):
        pltpu.async_copy(rows.at[b], acc_sh.at[dbuf.at[b]], ssem.at[b],
                         add=True)

    def wait_scatter(b):
        pltpu.make_async_copy(rows.at[b], acc_sh.at[dbuf.at[b]],
                              ssem.at[b]).wait()

    issue_idx(0, 0)
    issue_idx(1, 1)
    wait_idx(0)
    issue_gather(0)

    def group(g, carry):
        for b in range(3):
            i = g * 3 + b
            nb1 = (b + 1) % 3
            nb2 = (b + 2) % 3
            wait_gather(b)
            issue_scatter(b)

            @pl.when(i >= 1)
            def _():
                wait_scatter(nb2)

            @pl.when(i + 2 < NFULL)
            def _():
                issue_idx(i + 2, nb2)

            @pl.when(i + 1 < NFULL)
            def _():
                wait_idx(nb1)
                issue_gather(nb1)

        return carry

    lax.fori_loop(0, NFULL // 3, group, 0)
    wait_scatter((NFULL - 1) % 3)

    off = base + NFULL * CH
    pltpu.sync_copy(s_hbm.at[pl.ds(off, TAIL)], sbuf_t)
    pltpu.sync_copy(d_hbm.at[pl.ds(off, TAIL)], dbuf_t)
    pltpu.async_copy(y_hbm.at[sbuf_t], rows.at[0, pl.ds(0, TAIL)], tsem).wait()
    pltpu.sync_copy(rows.at[0, pl.ds(0, TAIL)], acc_sh.at[dbuf_t], add=True)
    plsc.subcore_barrier()
    pltpu.sync_copy(acc_sh.at[pl.ds(s * ZR, ZR)],
                    out_hbm.at[c, pl.ds(s * ZR, ZR)])

    @pl.when(s == NS - 1)
    def _():
        pltpu.sync_copy(acc_sh.at[pl.ds(NS * ZR, ZREM)],
                        out_hbm.at[c, pl.ds(NS * ZR, ZREM)])


def _agg_partials(y, s, d):
    f = pl.kernel(
        _agg_body,
        out_type=jax.ShapeDtypeStruct((NC, N, H), jnp.float32),
        mesh=_MESH,
        scratch_types=[
            pltpu.VMEM_SHARED((N, H), jnp.float32),
            pltpu.VMEM((3, CH), jnp.int32),
            pltpu.VMEM((3, CH), jnp.int32),
            pltpu.VMEM((3, CH, H), jnp.float32),
            pltpu.VMEM((TAIL,), jnp.int32),
            pltpu.VMEM((TAIL,), jnp.int32),
            pltpu.SemaphoreType.DMA((3,)),
            pltpu.SemaphoreType.DMA((3,)),
            pltpu.SemaphoreType.DMA((3,)),
            pltpu.SemaphoreType.DMA,
        ],
    )
    return f(y, s, d)


PCH = 64
NPCH = (N + PCH - 1) // PCH
PTAIL = N - (NPCH - 1) * PCH


def _pool_body(h_hbm, b_hbm, mx_hbm, sm_hbm, cn_hbm,
               rowbuf, bbuf, rowbuf_t, bbuf_t, mx, sm, cn):
    c = lax.axis_index("c")
    s = lax.axis_index("s")
    wid = c * NS + s

    neg_inf = jnp.full((16,), -jnp.inf, dtype=jnp.float32)

    def initrow(i, carry):
        for j in range(H // 16):
            mx[i, pl.ds(j * 16, 16)] = neg_inf
            sm[i, pl.ds(j * 16, 16)] = jnp.zeros((16,), jnp.float32)
        cn[i, pl.ds(0, 16)] = jnp.zeros((16,), jnp.float32)
        return carry

    lax.fori_loop(0, G, initrow, 0)

    ones16 = jnp.ones((16,), jnp.float32)

    def accum_row(rb, bb, i):
        g = bb[pl.ds(i, 16)][0]
        plsc.addupdate(cn.at[g], ones16)
        for j in range(H // 16):
            r = rb[i, pl.ds(j * 16, 16)]
            plsc.addupdate(sm.at[g, pl.ds(j * 16, 16)], r)
            m = mx[g, pl.ds(j * 16, 16)]
            mx[g, pl.ds(j * 16, 16)] = jnp.maximum(m, r)

    def do_chunk(jj, carry):
        k = wid + jj * NW

        @pl.when(k < NPCH - 1)
        def _():
            pltpu.sync_copy(h_hbm.at[pl.ds(k * PCH, PCH)], rowbuf)
            pltpu.sync_copy(b_hbm.at[pl.ds(k * PCH, PCH)], bbuf.at[pl.ds(0, PCH)])

            def row(i, cc):
                accum_row(rowbuf, bbuf, i)
                return cc

            lax.fori_loop(0, PCH, row, 0)

        @pl.when(k == NPCH - 1)
        def _():
            pltpu.sync_copy(h_hbm.at[pl.ds((NPCH - 1) * PCH, PTAIL)], rowbuf_t)
            pltpu.sync_copy(b_hbm.at[pl.ds((NPCH - 1) * PCH, PTAIL)],
                            bbuf_t.at[pl.ds(0, PTAIL)])

            def row(i, cc):
                accum_row(rowbuf_t, bbuf_t, i)
                return cc

            lax.fori_loop(0, PTAIL, row, 0)

        return carry

    lax.fori_loop(0, (NPCH + NW - 1) // NW, do_chunk, 0)

    pltpu.sync_copy(mx, mx_hbm.at[wid])
    pltpu.sync_copy(sm, sm_hbm.at[wid])
    pltpu.sync_copy(cn, cn_hbm.at[wid])


def _pool_partials(h, batch):
    f = pl.kernel(
        _pool_body,
        out_type=(
            jax.ShapeDtypeStruct((NW, G, H), jnp.float32),
            jax.ShapeDtypeStruct((NW, G, H), jnp.float32),
            jax.ShapeDtypeStruct((NW, G, 16), jnp.float32),
        ),
        mesh=_MESH,
        scratch_types=[
            pltpu.VMEM((PCH, H), jnp.float32),
            pltpu.VMEM((PCH + 16,), jnp.int32),
            pltpu.VMEM((PTAIL, H), jnp.float32),
            pltpu.VMEM((PTAIL + 16,), jnp.int32),
            pltpu.VMEM((G, H), jnp.float32),
            pltpu.VMEM((G, H), jnp.float32),
            pltpu.VMEM((G, 16), jnp.float32),
        ],
    )
    return f(h, batch)


RB = 1000


def _dinv_from_parts(deg_parts):
    deg = 1.0 + deg_parts[0, :, 0] + deg_parts[1, :, 0]
    return 1.0 / jnp.sqrt(deg)


def _xw_body(x_ref, w_ref, y_ref):
    y_ref[...] = jnp.dot(x_ref[...], w_ref[...],
                         preferred_element_type=jnp.float32)


def _xw_raw(x, w):
    fin = x.shape[1]
    return pl.pallas_call(
        _xw_body,
        grid=(N // RB,),
        in_specs=[
            pl.BlockSpec((RB, fin), lambda i: (i, 0)),
            pl.BlockSpec((fin, H), lambda i: (0, 0)),
        ],
        out_specs=pl.BlockSpec((RB, H), lambda i: (i, 0)),
        out_shape=jax.ShapeDtypeStruct((N, H), jnp.float32),
    )(x, w)


def _scale_body(xw_ref, degp_ref, y_ref):
    dinv = _dinv_from_parts(degp_ref[...])
    y_ref[...] = dinv[:, None] * xw_ref[...]


def _scale(xw, deg_parts):
    return pl.pallas_call(
        _scale_body,
        grid=(N // RB,),
        in_specs=[
            pl.BlockSpec((RB, H), lambda i: (i, 0)),
            pl.BlockSpec((NC, RB, 16), lambda i: (0, i, 0)),
        ],
        out_specs=pl.BlockSpec((RB, H), lambda i: (i, 0)),
        out_shape=jax.ShapeDtypeStruct((N, H), jnp.float32),
    )(xw, deg_parts)


def _combine_mm_body(aggp_ref, y_ref, degp_ref, b_ref, w_ref, out_ref):
    dinv = _dinv_from_parts(degp_ref[...])
    h = aggp_ref[0] + aggp_ref[1] + y_ref[...]
    h = jax.nn.relu(dinv[:, None] * h + b_ref[...])
    hw = jnp.dot(h, w_ref[...], preferred_element_type=jnp.float32)
    out_ref[...] = dinv[:, None] * hw


def _combine_matmul(aggp, y, deg_parts, b, w):
    return pl.pallas_call(
        _combine_mm_body,
        grid=(N // RB,),
        in_specs=[
            pl.BlockSpec((NC, RB, H), lambda i: (0, i, 0)),
            pl.BlockSpec((RB, H), lambda i: (i, 0)),
            pl.BlockSpec((NC, RB, 16), lambda i: (0, i, 0)),
            pl.BlockSpec((1, H), lambda i: (0, 0)),
            pl.BlockSpec((H, H), lambda i: (0, 0)),
        ],
        out_specs=pl.BlockSpec((RB, H), lambda i: (i, 0)),
        out_shape=jax.ShapeDtypeStruct((N, H), jnp.float32),
    )(aggp, y, deg_parts, b, w)


def _combine_body(aggp_ref, y_ref, degp_ref, b_ref, out_ref):
    dinv = _dinv_from_parts(degp_ref[...])
    h = aggp_ref[0] + aggp_ref[1] + y_ref[...]
    out_ref[...] = jax.nn.relu(dinv[:, None] * h + b_ref[...])


def _combine(aggp, y, deg_parts, b):
    return pl.pallas_call(
        _combine_body,
        grid=(N // RB,),
        in_specs=[
            pl.BlockSpec((NC, RB, H), lambda i: (0, i, 0)),
            pl.BlockSpec((RB, H), lambda i: (i, 0)),
            pl.BlockSpec((NC, RB, 16), lambda i: (0, i, 0)),
            pl.BlockSpec((1, H), lambda i: (0, 0)),
        ],
        out_specs=pl.BlockSpec((RB, H), lambda i: (i, 0)),
        out_shape=jax.ShapeDtypeStruct((N, H), jnp.float32),
    )(aggp, y, deg_parts, b)


def _final_body(mxp_ref, smp_ref, cnp_ref, w_ref, b_ref, out_ref):
    mx = jnp.max(mxp_ref[...], axis=0)
    sm = jnp.sum(smp_ref[...], axis=0)
    cnt = jnp.sum(cnp_ref[..., 0], axis=0)
    mean = sm / jnp.maximum(cnt, 1.0)[:, None]
    z = jnp.concatenate([mx, mean], axis=1)
    z = jnp.dot(z, w_ref[...], preferred_element_type=jnp.float32) + b_ref[...]
    m = jnp.max(z, axis=1, keepdims=True)
    lse = jnp.log(jnp.sum(jnp.exp(z - m), axis=1, keepdims=True)) + m
    out_ref[...] = z - lse


def _final(mxp, smp, cnp, lin_W, lin_b):
    return pl.pallas_call(
        _final_body,
        out_shape=jax.ShapeDtypeStruct((G, C), jnp.float32),
    )(mxp, smp, cnp, lin_W, lin_b.reshape(1, C))


def kernel(x, edge_index, batch, W1, b1, W2, b2, lin_W, lin_b):
    s = edge_index[0]
    d = edge_index[1]
    xwr = _xw_raw(x, W1)
    deg_parts = _deg_partials(d)
    y1 = _scale(xwr, deg_parts)
    agg1 = _agg_partials(y1, s, d)
    y2 = _combine_matmul(agg1, y1, deg_parts, b1.reshape(1, H), W2)
    agg2 = _agg_partials(y2, s, d)
    h2 = _combine(agg2, y2, deg_parts, b2.reshape(1, H))
    mxp, smp, cnp = _pool_partials(h2, batch)
    return _final(mxp, smp, cnp, lin_W, lin_b)


# speedup vs baseline: 1.0789x; 1.0021x over previous
import functools

import jax
import jax.numpy as jnp
from jax import lax
from jax.experimental import pallas as pl
from jax.experimental.pallas import tpu as pltpu
from jax.experimental.pallas import tpu_sc as plsc

N = 10000
E = 320000
H = 128
G = 64
C = 10

NC = 2
NS = 16
NW = NC * NS

EW = E // NW
CH = 128
NFULL = EW // CH
TAIL = EW - NFULL * CH
ZR = 624
ZREM = N - NS * ZR

_MESH = plsc.VectorSubcoreMesh(core_axis_name="c", subcore_axis_name="s",
                               num_cores=NC, num_subcores=NS)


def _zero_vmem_rows(ref, nrows, width):
    nch = width // 16

    def row(i, carry):
        for j in range(nch):
            ref[i, pl.ds(j * 16, 16)] = jnp.zeros((16,), jnp.float32)
        return carry

    lax.fori_loop(0, nrows, row, 0)


def _deg_body(d_hbm, out_hbm, deg_sh, ones_v, dbuf, dbuf_t, zb, isem, ssem):
    c = lax.axis_index("c")
    s = lax.axis_index("s")
    wid = c * NS + s

    def setrow(i, carry):
        ones_v[i, pl.ds(0, 16)] = jnp.ones((16,), jnp.float32)
        zb[i, pl.ds(0, 16)] = jnp.zeros((16,), jnp.float32)
        return carry

    lax.fori_loop(0, CH, setrow, 0)
    for k, sz in ((0, 128), (128, 128), (256, 128), (384, 128), (512, 112)):
        pltpu.sync_copy(zb.at[pl.ds(0, sz)],
                        deg_sh.at[pl.ds(s * ZR + k, sz)])

    @pl.when(s == NS - 1)
    def _():
        pltpu.sync_copy(zb.at[pl.ds(0, ZREM)],
                        deg_sh.at[pl.ds(NS * ZR, ZREM)])

    plsc.subcore_barrier()

    base = wid * EW

    def issue_idx(i, b):
        pltpu.async_copy(d_hbm.at[pl.ds(base + i * CH, CH)], dbuf.at[b],
                         isem.at[b])

    def wait_idx(b):
        pltpu.make_async_copy(d_hbm.at[pl.ds(base, CH)], dbuf.at[b],
                              isem.at[b]).wait()

    def issue_scatter(b):
        pltpu.async_copy(ones_v, deg_sh.at[dbuf.at[b]], ssem.at[b], add=True)

    def wait_scatter(b):
        pltpu.make_async_copy(ones_v, deg_sh.at[dbuf.at[b]], ssem.at[b]).wait()

    issue_idx(0, 0)
    issue_idx(1, 1)

    def group(g, carry):
        for b in range(3):
            i = g * 3 + b
            wait_idx(b)
            issue_scatter(b)
            nb2 = (b + 2) % 3

            @pl.when(i >= 1)
            def _():
                wait_scatter(nb2)

            @pl.when(i + 2 < NFULL)
            def _():
                issue_idx(i + 2, nb2)

        return carry

    lax.fori_loop(0, NFULL // 3, group, 0)
    wait_scatter((NFULL - 1) % 3)
    pltpu.sync_copy(d_hbm.at[pl.ds(base + NFULL * CH, TAIL)], dbuf_t)
    pltpu.sync_copy(ones_v.at[pl.ds(0, TAIL)], deg_sh.at[dbuf_t], add=True)
    plsc.subcore_barrier()
    pltpu.sync_copy(deg_sh.at[pl.ds(s * ZR, ZR)],
                    out_hbm.at[c, pl.ds(s * ZR, ZR)])

    @pl.when(s == NS - 1)
    def _():
        pltpu.sync_copy(deg_sh.at[pl.ds(NS * ZR, ZREM)],
                        out_hbm.at[c, pl.ds(NS * ZR, ZREM)])


def _deg_partials(d):
    f = pl.kernel(
        _deg_body,
        out_type=jax.ShapeDtypeStruct((NC, N, 16), jnp.float32),
        mesh=_MESH,
        scratch_types=[
            pltpu.VMEM_SHARED((N, 16), jnp.float32),
            pltpu.VMEM((CH, 16), jnp.float32),
            pltpu.VMEM((3, CH), jnp.int32),
            pltpu.VMEM((TAIL,), jnp.int32),
            pltpu.VMEM((CH, 16), jnp.float32),
            pltpu.SemaphoreType.DMA((3,)),
            pltpu.SemaphoreType.DMA((3,)),
        ],
    )
    return f(d)


def _agg_body(y_hbm, s_hbm, d_hbm, out_hbm, acc_sh, sbuf, dbuf, rows,
              sbuf_t, dbuf_t, isem_s, isem_d, gsem, ssem, tsem):
    c = lax.axis_index("c")
    s = lax.axis_index("s")
    wid = c * NS + s

    def zrow(i, carry):
        for j in range(H // 16):
            rows[0, i, pl.ds(j * 16, 16)] = jnp.zeros((16,), jnp.float32)
        return carry

    lax.fori_loop(0, 128, zrow, 0)
    for k, sz in ((0, 128), (128, 128), (256, 128), (384, 128), (512, 112)):
        pltpu.sync_copy(rows.at[0, pl.ds(0, sz)],
                        acc_sh.at[pl.ds(s * ZR + k, sz)])

    @pl.when(s == NS - 1)
    def _():
        pltpu.sync_copy(rows.at[0, pl.ds(0, ZREM)],
                        acc_sh.at[pl.ds(NS * ZR, ZREM)])

    plsc.subcore_barrier()

    base = wid * EW

    def issue_idx(i, b):
        off = base + i * CH
        pltpu.async_copy(s_hbm.at[pl.ds(off, CH)], sbuf.at[b], isem_s.at[b])
        pltpu.async_copy(d_hbm.at[pl.ds(off, CH)], dbuf.at[b], isem_d.at[b])

    def wait_idx(b):
        pltpu.make_async_copy(s_hbm.at[pl.ds(base, CH)], sbuf.at[b],
                              isem_s.at[b]).wait()
        pltpu.make_async_copy(d_hbm.at[pl.ds(base, CH)], dbuf.at[b],
                              isem_d.at[b]).wait()

    def issue_gather(b):
        pltpu.async_copy(y_hbm.at[sbuf.at[b]], rows.at[b], gsem.at[b])

    def wait_gather(b):
        pltpu.make_async_copy(y_hbm.at[sbuf.at[b]], rows.at[b],
                              gsem.at[b]).wait()

    def issue_scatter(b):
        pltpu.async_copy(rows.at[b], acc_sh.at[dbuf.at[b]], ssem.at[b],
                         add=True)

    def wait_scatter(b):
        pltpu.make_async_copy(rows.at[b], acc_sh.at[dbuf.at[b]],
                              ssem.at[b]).wait()

    issue_idx(0, 0)
    issue_idx(1, 1)
    wait_idx(0)
    issue_gather(0)

    def group(g, carry):
        for b in range(3):
            i = g * 3 + b
            nb1 = (b + 1) % 3
            nb2 = (b + 2) % 3
            wait_gather(b)
            issue_scatter(b)

            @pl.when(i >= 1)
            def _():
                wait_scatter(nb2)

            @pl.when(i + 2 < NFULL)
            def _():
                issue_idx(i + 2, nb2)

            @pl.when(i + 1 < NFULL)
            def _():
                wait_idx(nb1)
                issue_gather(nb1)

        return carry

    lax.fori_loop(0, NFULL // 3, group, 0)
    wait_scatter((NFULL - 1) % 3)

    off = base + NFULL * CH
    pltpu.sync_copy(s_hbm.at[pl.ds(off, TAIL)], sbuf_t)
    pltpu.sync_copy(d_hbm.at[pl.ds(off, TAIL)], dbuf_t)
    pltpu.async_copy(y_hbm.at[sbuf_t], rows.at[0, pl.ds(0, TAIL)], tsem).wait()
    pltpu.sync_copy(rows.at[0, pl.ds(0, TAIL)], acc_sh.at[dbuf_t], add=True)
    plsc.subcore_barrier()
    pltpu.sync_copy(acc_sh.at[pl.ds(s * ZR, ZR)],
                    out_hbm.at[c, pl.ds(s * ZR, ZR)])

    @pl.when(s == NS - 1)
    def _():
        pltpu.sync_copy(acc_sh.at[pl.ds(NS * ZR, ZREM)],
                        out_hbm.at[c, pl.ds(NS * ZR, ZREM)])


def _agg_partials(y, s, d):
    f = pl.kernel(
        _agg_body,
        out_type=jax.ShapeDtypeStruct((NC, N, H), jnp.float32),
        mesh=_MESH,
        scratch_types=[
            pltpu.VMEM_SHARED((N, H), jnp.float32),
            pltpu.VMEM((3, CH), jnp.int32),
            pltpu.VMEM((3, CH), jnp.int32),
            pltpu.VMEM((3, CH, H), jnp.float32),
            pltpu.VMEM((TAIL,), jnp.int32),
            pltpu.VMEM((TAIL,), jnp.int32),
            pltpu.SemaphoreType.DMA((3,)),
            pltpu.SemaphoreType.DMA((3,)),
            pltpu.SemaphoreType.DMA((3,)),
            pltpu.SemaphoreType.DMA((3,)),
            pltpu.SemaphoreType.DMA,
        ],
    )
    return f(y, s, d)


PCH = 64
NPCH = (N + PCH - 1) // PCH
PTAIL = N - (NPCH - 1) * PCH


def _pool_body(h_hbm, b_hbm, mx_hbm, sm_hbm, cn_hbm,
               rowbuf, bbuf, rowbuf_t, bbuf_t, mx, sm, cn):
    c = lax.axis_index("c")
    s = lax.axis_index("s")
    wid = c * NS + s

    neg_inf = jnp.full((16,), -jnp.inf, dtype=jnp.float32)

    def initrow(i, carry):
        for j in range(H // 16):
            mx[i, pl.ds(j * 16, 16)] = neg_inf
            sm[i, pl.ds(j * 16, 16)] = jnp.zeros((16,), jnp.float32)
        cn[i, pl.ds(0, 16)] = jnp.zeros((16,), jnp.float32)
        return carry

    lax.fori_loop(0, G, initrow, 0)

    ones16 = jnp.ones((16,), jnp.float32)

    def accum_row(rb, bb, i):
        g = bb[pl.ds(i, 16)][0]
        plsc.addupdate(cn.at[g], ones16)
        for j in range(H // 16):
            r = rb[i, pl.ds(j * 16, 16)]
            plsc.addupdate(sm.at[g, pl.ds(j * 16, 16)], r)
            m = mx[g, pl.ds(j * 16, 16)]
            mx[g, pl.ds(j * 16, 16)] = jnp.maximum(m, r)

    def do_chunk(jj, carry):
        k = wid + jj * NW

        @pl.when(k < NPCH - 1)
        def _():
            pltpu.sync_copy(h_hbm.at[pl.ds(k * PCH, PCH)], rowbuf)
            pltpu.sync_copy(b_hbm.at[pl.ds(k * PCH, PCH)], bbuf.at[pl.ds(0, PCH)])

            def row(i, cc):
                accum_row(rowbuf, bbuf, i)
                return cc

            lax.fori_loop(0, PCH, row, 0)

        @pl.when(k == NPCH - 1)
        def _():
            pltpu.sync_copy(h_hbm.at[pl.ds((NPCH - 1) * PCH, PTAIL)], rowbuf_t)
            pltpu.sync_copy(b_hbm.at[pl.ds((NPCH - 1) * PCH, PTAIL)],
                            bbuf_t.at[pl.ds(0, PTAIL)])

            def row(i, cc):
                accum_row(rowbuf_t, bbuf_t, i)
                return cc

            lax.fori_loop(0, PTAIL, row, 0)

        return carry

    lax.fori_loop(0, (NPCH + NW - 1) // NW, do_chunk, 0)

    pltpu.sync_copy(mx, mx_hbm.at[wid])
    pltpu.sync_copy(sm, sm_hbm.at[wid])
    pltpu.sync_copy(cn, cn_hbm.at[wid])


def _pool_partials(h, batch):
    f = pl.kernel(
        _pool_body,
        out_type=(
            jax.ShapeDtypeStruct((NW, G, H), jnp.float32),
            jax.ShapeDtypeStruct((NW, G, H), jnp.float32),
            jax.ShapeDtypeStruct((NW, G, 16), jnp.float32),
        ),
        mesh=_MESH,
        scratch_types=[
            pltpu.VMEM((PCH, H), jnp.float32),
            pltpu.VMEM((PCH + 16,), jnp.int32),
            pltpu.VMEM((PTAIL, H), jnp.float32),
            pltpu.VMEM((PTAIL + 16,), jnp.int32),
            pltpu.VMEM((G, H), jnp.float32),
            pltpu.VMEM((G, H), jnp.float32),
            pltpu.VMEM((G, 16), jnp.float32),
        ],
    )
    return f(h, batch)


RB = 1000


def _dinv_from_parts(deg_parts):
    deg = 1.0 + deg_parts[0, :, 0] + deg_parts[1, :, 0]
    return 1.0 / jnp.sqrt(deg)


def _xw_body(x_ref, w_ref, degp_ref, y_ref):
    dinv = _dinv_from_parts(degp_ref[...])
    xw = jnp.dot(x_ref[...], w_ref[...], preferred_element_type=jnp.float32)
    y_ref[...] = dinv[:, None] * xw


def _xw_scaled(x, w, deg_parts):
    fin = x.shape[1]
    return pl.pallas_call(
        _xw_body,
        grid=(N // RB,),
        in_specs=[
            pl.BlockSpec((RB, fin), lambda i: (i, 0)),
            pl.BlockSpec((fin, H), lambda i: (0, 0)),
            pl.BlockSpec((NC, RB, 16), lambda i: (0, i, 0)),
        ],
        out_specs=pl.BlockSpec((RB, H), lambda i: (i, 0)),
        out_shape=jax.ShapeDtypeStruct((N, H), jnp.float32),
    )(x, w, deg_parts)


def _combine_mm_body(aggp_ref, y_ref, degp_ref, b_ref, w_ref, out_ref):
    dinv = _dinv_from_parts(degp_ref[...])
    h = aggp_ref[0] + aggp_ref[1] + y_ref[...]
    h = jax.nn.relu(dinv[:, None] * h + b_ref[...])
    hw = jnp.dot(h, w_ref[...], preferred_element_type=jnp.float32)
    out_ref[...] = dinv[:, None] * hw


def _combine_matmul(aggp, y, deg_parts, b, w):
    return pl.pallas_call(
        _combine_mm_body,
        grid=(N // RB,),
        in_specs=[
            pl.BlockSpec((NC, RB, H), lambda i: (0, i, 0)),
            pl.BlockSpec((RB, H), lambda i: (i, 0)),
            pl.BlockSpec((NC, RB, 16), lambda i: (0, i, 0)),
            pl.BlockSpec((1, H), lambda i: (0, 0)),
            pl.BlockSpec((H, H), lambda i: (0, 0)),
        ],
        out_specs=pl.BlockSpec((RB, H), lambda i: (i, 0)),
        out_shape=jax.ShapeDtypeStruct((N, H), jnp.float32),
    )(aggp, y, deg_parts, b, w)


def _combine_body(aggp_ref, y_ref, degp_ref, b_ref, out_ref):
    dinv = _dinv_from_parts(degp_ref[...])
    h = aggp_ref[0] + aggp_ref[1] + y_ref[...]
    out_ref[...] = jax.nn.relu(dinv[:, None] * h + b_ref[...])


def _combine(aggp, y, deg_parts, b):
    return pl.pallas_call(
        _combine_body,
        grid=(N // RB,),
        in_specs=[
            pl.BlockSpec((NC, RB, H), lambda i: (0, i, 0)),
            pl.BlockSpec((RB, H), lambda i: (i, 0)),
            pl.BlockSpec((NC, RB, 16), lambda i: (0, i, 0)),
            pl.BlockSpec((1, H), lambda i: (0, 0)),
        ],
        out_specs=pl.BlockSpec((RB, H), lambda i: (i, 0)),
        out_shape=jax.ShapeDtypeStruct((N, H), jnp.float32),
    )(aggp, y, deg_parts, b)


def _final_body(mxp_ref, smp_ref, cnp_ref, w_ref, b_ref, out_ref):
    mx = jnp.max(mxp_ref[...], axis=0)
    sm = jnp.sum(smp_ref[...], axis=0)
    cnt = jnp.sum(cnp_ref[..., 0], axis=0)
    mean = sm / jnp.maximum(cnt, 1.0)[:, None]
    z = jnp.concatenate([mx, mean], axis=1)
    z = jnp.dot(z, w_ref[...], preferred_element_type=jnp.float32) + b_ref[...]
    m = jnp.max(z, axis=1, keepdims=True)
    lse = jnp.log(jnp.sum(jnp.exp(z - m), axis=1, keepdims=True)) + m
    out_ref[...] = z - lse


def _final(mxp, smp, cnp, lin_W, lin_b):
    return pl.pallas_call(
        _final_body,
        out_shape=jax.ShapeDtypeStruct((G, C), jnp.float32),
    )(mxp, smp, cnp, lin_W, lin_b.reshape(1, C))


def kernel(x, edge_index, batch, W1, b1, W2, b2, lin_W, lin_b):
    s = edge_index[0]
    d = edge_index[1]
    deg_parts = _deg_partials(d)
    y1 = _xw_scaled(x, W1, deg_parts)
    agg1 = _agg_partials(y1, s, d)
    y2 = _combine_matmul(agg1, y1, deg_parts, b1.reshape(1, H), W2)
    agg2 = _agg_partials(y2, s, d)
    h2 = _combine(agg2, y2, deg_parts, b2.reshape(1, H))
    mxp, smp, cnp = _pool_partials(h2, batch)
    return _final(mxp, smp, cnp, lin_W, lin_b)
